# baseline (speedup 1.0000x reference)
import functools

import jax
import jax.numpy as jnp
from jax.experimental import pallas as pl
from jax.experimental.pallas import tpu as pltpu

H, W = 32, 256
HW = H * W
N = 4096
C = 64
NSAMPLE = 4
NSAMPLE_Q = 6
DIST2 = 100.0

Q1 = 512
G1 = HW // Q1
Q2 = 1024
G2 = HW // Q2
PAD = 264
DWS = (-2, -1, 0, 1, 2)


def _lrelu(x):
    return jnp.where(x > 0, x, 0.1 * x)


def _mm(a, b):
    return jax.lax.dot_general(a, b, (((1,), (0,)), ((), ())),
                               preferred_element_type=jnp.float32)


def _norm_rows(x, n):
    m = jnp.mean(x, axis=1, keepdims=True)
    s = jnp.sqrt(jnp.sum((x - m) ** 2, axis=1, keepdims=True) / (n - 1.0))
    return (x - m) / jnp.maximum(s, 1e-12)


def _prologue_body(f2x_ref, f2p_ref, f2cat_ref, knrow_ref):
    f2x = f2x_ref[...]
    f2n = _norm_rows(f2p_ref[...], C)
    f2cat_ref[...] = jnp.concatenate([f2n, f2x], axis=1)
    kn = jnp.sum(f2x * f2x, axis=1, keepdims=True)
    knrow_ref[...] = jax.lax.transpose(kn, (1, 0))


def _stage1_body(wx_ref, lz_ref, wp_ref, f2x_ref, f2cat_ref, knrow_ref,
                 w1a_ref, w1bp_ref, w1c_ref, b1_ref, w11_ref, b11_ref,
                 wpa_ref, wpbp_ref, bp_ref,
                 w2a_ref, w2b_ref, b2_ref, w21_ref, b21_ref,
                 out_ref):
    wxq = wx_ref[...]
    wxyz = wxq * lz_ref[...]
    f2x = f2x_ref[...]
    f2cat = f2cat_ref[...]

    qn = jnp.sum(wxq * wxq, axis=1, keepdims=True)
    qk = jax.lax.dot_general(wxq, f2x, (((1,), (1,)), ((), ())),
                             preferred_element_type=jnp.float32)
    d2 = (qn + knrow_ref[...]) - 2.0 * qk

    pn = _norm_rows(wp_ref[...], C)

    wxa = _mm(wxyz, w1a_ref[...])
    wxpi = _mm(wxyz, wpa_ref[...])

    iota = jax.lax.broadcasted_iota(jnp.int32, (Q1, N), 1)
    d = d2
    feats = []
    pics = []
    for _ in range(NSAMPLE_Q):
        idx = jnp.argmin(d, axis=1, keepdims=True)
        ohb = iota == idx
        oh = ohb.astype(jnp.float32)
        d = jnp.where(ohb, 1e30, d)
        g = _mm(oh, f2cat)
        pj = g[:, :C]
        pre1 = wxa + _mm(g, w1bp_ref[...]) + _mm(pn * pj, w1c_ref[...]) \
            + b1_ref[...]
        h1 = _lrelu(pre1)
        feat = _lrelu(_mm(h1, w11_ref[...]) + b11_ref[...])
        enc = _lrelu(wxpi + _mm(g, wpbp_ref[...]) + bp_ref[...])
        h2 = _lrelu(_mm(enc, w2a_ref[...]) + _mm(feat, w2b_ref[...])
                    + b2_ref[...])
        pic = _lrelu(_mm(h2, w21_ref[...]) + b21_ref[...])
        feats.append(feat)
        pics.append(pic)

    mx = pics[0]
    for p in pics[1:]:
        mx = jnp.maximum(mx, p)
    ssum = None
    acc = None
    for p, f in zip(pics, feats):
        e = jnp.exp(p - mx)
        ssum = e if ssum is None else ssum + e
        t = e * f
        acc = t if acc is None else acc + t
    out_ref[...] = acc / ssum


def _repack_body(xin_ref, win_ref, xpack_ref, wpack_ref):
    L = HW + 2 * PAD
    xv = xin_ref[...]
    wv = win_ref[...]
    xpack_ref[...] = jnp.concatenate(
        [xv[8 + dw:8 + dw + L, :] for dw in DWS], axis=1)
    wpack_ref[...] = jnp.concatenate(
        [wv[8 + dw:8 + dw + L, :] for dw in DWS], axis=1)


def _stage2_body(ctr_ref, wxyz_ref, wp_ref,
                 xpack_ref, wpack_ref, fpad_ref,
                 wpca_ref, wpcb_ref, wpcc_ref, wpcd_ref, bpc_ref,
                 w2ba_ref, w2bb_ref, w2bc_ref, b2b_ref, w2b1_ref, b2b1_ref,
                 out_ref):
    i = pl.program_id(0)
    base = i * Q2 + PAD

    ctr = ctr_ref[...]
    wxyzq = wxyz_ref[...]

    pidx = jax.lax.broadcasted_iota(jnp.int32, (Q2, 1), 0)
    col = pidx % W
    row = i * (Q2 // W) + pidx // W

    f32 = jnp.float32
    i3r = jax.lax.broadcasted_iota(jnp.int32, (3, 15), 0)
    i15c = jax.lax.broadcasted_iota(jnp.int32, (3, 15), 1)
    Bm = (i15c % 3 == i3r).astype(f32)
    Bmt = jax.lax.transpose(Bm, (1, 0))
    i15a = jax.lax.broadcasted_iota(jnp.int32, (15, 15), 0)
    i15b = jax.lax.broadcasted_iota(jnp.int32, (15, 15), 1)
    SP = (i15a == (i15b // 3) * 3).astype(f32)
    i15e = jax.lax.broadcasted_iota(jnp.int32, (15, 320), 0)
    i320 = jax.lax.broadcasted_iota(jnp.int32, (15, 320), 1)
    E320 = (i15e == (i320 // 64) * 3).astype(f32)

    lane15 = jax.lax.broadcasted_iota(jnp.int32, (1, 15), 1)
    dwl = lane15 // 3 - 2
    okw15 = (col + dwl >= 0) & (col + dwl < W)
    is3 = lane15 % 3 == 0

    ctrb = jnp.concatenate([ctr] * 5, axis=1)

    fwide = fpad_ref[pl.ds(i * Q2, Q2 + 2 * W + 16), :]
    fcols = [fwide[8 + dw:8 + dw + Q2 + 2 * W, :] for dw in DWS]

    d2s = []
    wslices = []
    fslices = []
    for dh in (-1, 0, 1):
        sl = pl.ds(base + dh * W, Q2)
        xs = xpack_ref[sl, :]
        e = (xs - ctrb) ** 2
        e1 = jnp.concatenate([e[:, 1:15], e[:, 0:1]], axis=1)
        e2 = jnp.concatenate([e[:, 2:15], e[:, 0:2]], axis=1)
        dd = (e + e1) + e2
        okh = (row + dh >= 0) & (row + dh < H)
        d2s.append(jnp.where(is3 & okh & okw15, dd, 1e10))
        wslices.append(wpack_ref[sl, :])
        r0 = (dh + 1) * W
        fslices.append([fc[r0:r0 + Q2, :] for fc in fcols])

    gx = []
    gf = []
    valid = []
    for _ in range(NSAMPLE):
        m = jnp.min(d2s[0], axis=1, keepdims=True)
        for o in (1, 2):
            m = jnp.minimum(m, jnp.min(d2s[o], axis=1, keepdims=True))
        gxk = None
        gfk = None
        nds = []
        for t in range(3):
            eq = d2s[t] == m
            nds.append(jnp.where(eq, 1e30, d2s[t]))
            s = eq.astype(f32)
            txk = _mm(s, SP) * wslices[t]
            gxk = txk if gxk is None else gxk + txk
            fs = fslices[t]
            ce = _mm(s, E320)
            tfk = ce[:, 0:64] * fs[0] + ce[:, 64:128] * fs[1] \
                + ce[:, 128:192] * fs[2] + ce[:, 192:256] * fs[3] \
                + ce[:, 256:320] * fs[4]
            gfk = tfk if gfk is None else gfk + tfk
        d2s = nds
        gx.append(_mm(gxk, Bmt))
        gf.append(gfk)
        valid.append((m < DIST2).astype(f32))

    ptsnew = _mm(wp_ref[...], w2bb_ref[...])
    wxenc = _mm(wxyzq, wpca_ref[...])

    pccs = []
    for k in range(NSAMPLE):
        diff = gx[k] - wxyzq
        euc = jnp.sqrt(jnp.sum(diff * diff, axis=1, keepdims=True) + 1e-20)
        enc = _lrelu(wxenc + _mm(gx[k], wpcb_ref[...])
                     + _mm(diff, wpcc_ref[...])
                     + euc * wpcd_ref[...] + bpc_ref[...])
        h = _lrelu(_mm(enc, w2ba_ref[...]) + ptsnew
                   + _mm(gf[k], w2bc_ref[...]) + b2b_ref[...])
        pcc = _lrelu(_mm(h, w2b1_ref[...]) + b2b1_ref[...])
        pccs.append(pcc * valid[k] + (-1e10) * (1.0 - valid[k]))

    mx = pccs[0]
    for p in pccs[1:]:
        mx = jnp.maximum(mx, p)
    ssum = None
    acc = None
    for p, g in zip(pccs, gf):
        e = jnp.exp(p - mx)
        ssum = e if ssum is None else ssum + e
        t = e * g
        acc = t if acc is None else acc + t
    out_ref[...] = acc / ssum


def _full_spec(shape):
    return pl.BlockSpec(shape, lambda i: tuple(0 for _ in shape))


def _row_spec(blk, c):
    return pl.BlockSpec((blk, c), lambda i: (i, 0))


@functools.partial(jax.jit, static_argnames=("interpret",))
def _run(xyz_proj_raw, warped_xyz, warped_points, f2_xyz, f2_points,
         lidar_z, params, interpret=False):
    wx = warped_xyz[0]
    wp = warped_points[0]
    lz = lidar_z[0]
    f2x = f2_xyz[0]
    f2p = f2_points[0]
    xp = xyz_proj_raw.reshape(HW, 3)

    w1_0 = params['mlp1_0_w'].T
    w1a = w1_0[0:3]
    w1bp = jnp.zeros((C + 3, 128), jnp.float32).at[C:].set(w1_0[3:6])
    w1c = w1_0[6:70]
    b1 = params['mlp1_0_b'][None, :]
    w11 = params['mlp1_1_w'].T
    b11 = params['mlp1_1_b'][None, :]
    wpi = params['pi_enc_w'].T
    wpa = wpi[0:3]
    wpbp = jnp.zeros((C + 3, 64), jnp.float32).at[C:].set(wpi[3:6])
    bp = params['pi_enc_b'][None, :]
    w2_0 = params['mlp2_0_w'].T
    w2a = w2_0[0:64]
    w2b = w2_0[64:128]
    b2 = params['mlp2_0_b'][None, :]
    w21 = params['mlp2_1_w'].T
    b21 = params['mlp2_1_b'][None, :]

    f2cat, knrow = pl.pallas_call(
        _prologue_body,
        in_specs=[pl.BlockSpec((N, 3), None), pl.BlockSpec((N, C), None)],
        out_specs=[pl.BlockSpec((N, C + 3), None), pl.BlockSpec((1, N), None)],
        out_shape=[jax.ShapeDtypeStruct((N, C + 3), jnp.float32),
                   jax.ShapeDtypeStruct((1, N), jnp.float32)],
        interpret=interpret,
    )(f2x, f2p)

    s1_out = pl.pallas_call(
        _stage1_body,
        grid=(G1,),
        in_specs=[
            _row_spec(Q1, 3), _row_spec(Q1, 1), _row_spec(Q1, C),
            _full_spec((N, 3)), _full_spec((N, C + 3)), _full_spec((1, N)),
            _full_spec(w1a.shape), _full_spec(w1bp.shape),
            _full_spec(w1c.shape), _full_spec(b1.shape),
            _full_spec(w11.shape), _full_spec(b11.shape),
            _full_spec(wpa.shape), _full_spec(wpbp.shape),
            _full_spec(bp.shape),
            _full_spec(w2a.shape), _full_spec(w2b.shape),
            _full_spec(b2.shape), _full_spec(w21.shape),
            _full_spec(b21.shape),
        ],
        out_specs=_row_spec(Q1, C),
        out_shape=jax.ShapeDtypeStruct((HW, C), jnp.float32),
        interpret=interpret,
    )(wx, lz, wp, f2x, f2cat, knrow, w1a, w1bp, w1c, b1, w11, b11,
      wpa, wpbp, bp, w2a, w2b, b2, w21, b21)

    wxyz = wx * lz

    xin = jnp.pad(xp, ((PAD + 8, PAD + 8), (0, 0)))
    win = jnp.pad(wxyz, ((PAD + 8, PAD + 8), (0, 0)))
    LP = HW + 2 * PAD
    xpack, wpack = pl.pallas_call(
        _repack_body,
        in_specs=[pl.BlockSpec(xin.shape, None), pl.BlockSpec(win.shape, None)],
        out_specs=[pl.BlockSpec((LP, 15), None), pl.BlockSpec((LP, 15), None)],
        out_shape=[jax.ShapeDtypeStruct((LP, 15), jnp.float32),
                   jax.ShapeDtypeStruct((LP, 15), jnp.float32)],
        interpret=interpret,
    )(xin, win)
    fpad = jnp.pad(s1_out, ((PAD, PAD), (0, 0)))

    wpc = params['pc_enc_w'].T
    wpca = wpc[0:3]
    wpcb = wpc[3:6]
    wpcc = wpc[6:9]
    wpcd = wpc[9:10]
    bpc = params['pc_enc_b'][None, :]
    w2b_0 = params['mlp2b_0_w'].T
    w2ba = w2b_0[0:64]
    w2bb = w2b_0[64:128]
    w2bc = w2b_0[128:192]
    b2b = params['mlp2b_0_b'][None, :]
    w2b1 = params['mlp2b_1_w'].T
    b2b1 = params['mlp2b_1_b'][None, :]

    out = pl.pallas_call(
        _stage2_body,
        grid=(G2,),
        in_specs=[
            _row_spec(Q2, 3), _row_spec(Q2, 3), _row_spec(Q2, C),
            _full_spec(xpack.shape), _full_spec(wpack.shape),
            _full_spec(fpad.shape),
            _full_spec(wpca.shape), _full_spec(wpcb.shape),
            _full_spec(wpcc.shape), _full_spec(wpcd.shape),
            _full_spec(bpc.shape),
            _full_spec(w2ba.shape), _full_spec(w2bb.shape),
            _full_spec(w2bc.shape), _full_spec(b2b.shape),
            _full_spec(w2b1.shape), _full_spec(b2b1.shape),
        ],
        out_specs=_row_spec(Q2, C),
        out_shape=jax.ShapeDtypeStruct((HW, C), jnp.float32),
        interpret=interpret,
    )(xp, wxyz, wp, xpack, wpack, fpad,
      wpca, wpcb, wpcc, wpcd, bpc, w2ba, w2bb, w2bc, b2b, w2b1, b2b1)

    return out.reshape(1, H, W, C)


def kernel(xyz_proj_raw, warped_xyz, warped_points, idx_n2, f2_xyz,
           f2_points, lidar_z, params):
    del idx_n2
    return _run(xyz_proj_raw, warped_xyz, warped_points, f2_xyz, f2_points,
                lidar_z, params)
